# baseline (speedup 1.0000x reference)
import jax
import jax.numpy as jnp
from jax import lax
from jax.experimental import pallas as pl
from jax.experimental.pallas import tpu as pltpu
from jax.experimental.pallas import tpu_sc as plsc

VOCAB_ROWS = 1000000
EMB_DIM = 64
N_TOKENS = 4096 * 200

_info = plsc.get_sparse_core_info()
_NC = _info.num_cores
_NS = _info.num_subcores
_NW = _NC * _NS
_ROWS_PER_W = N_TOKENS // _NW
_CHUNK = 128
_N_CHUNKS = _ROWS_PER_W // _CHUNK


def _emb_body(ids_hbm, table_hbm, out_hbm, idx_v, rows_v, sem):
    wid = lax.axis_index("s") * _NC + lax.axis_index("c")
    base = wid * _ROWS_PER_W
    pltpu.sync_copy(ids_hbm.at[pl.ds(base, _ROWS_PER_W)], idx_v)

    def step(j, carry):
        off = j * _CHUNK
        pltpu.async_copy(
            table_hbm.at[idx_v.at[pl.ds(off, _CHUNK)]], rows_v, sem
        ).wait()
        pltpu.sync_copy(rows_v, out_hbm.at[pl.ds(base + off, _CHUNK)])
        return carry

    lax.fori_loop(0, _N_CHUNKS, step, 0)


@jax.jit
def kernel(input_ids, table):
    ids = input_ids.reshape(-1).astype(jnp.int32)
    gather = pl.kernel(
        _emb_body,
        mesh=plsc.VectorSubcoreMesh(core_axis_name="c", subcore_axis_name="s"),
        out_type=jax.ShapeDtypeStruct((N_TOKENS, EMB_DIM), jnp.float32),
        scratch_types=[
            pltpu.VMEM((_ROWS_PER_W,), jnp.int32),
            pltpu.VMEM((_CHUNK, EMB_DIM), jnp.float32),
            pltpu.SemaphoreType.DMA,
        ],
        compiler_params=pltpu.CompilerParams(use_tc_tiling_on_sc=False),
    )
    out = gather(ids, table)
    return out.reshape(input_ids.shape + (EMB_DIM,))


# speedup vs baseline: 1.1139x; 1.1139x over previous
import jax
import jax.numpy as jnp
from jax import lax
from jax.experimental import pallas as pl
from jax.experimental.pallas import tpu as pltpu
from jax.experimental.pallas import tpu_sc as plsc

VOCAB_ROWS = 1000000
EMB_DIM = 64
N_TOKENS = 4096 * 200

_info = plsc.get_sparse_core_info()
_NC = _info.num_cores
_NS = _info.num_subcores
_NW = _NC * _NS
_ROWS_PER_W = N_TOKENS // _NW
_CHUNK = 256
_NB = 4
_N_CHUNKS = _ROWS_PER_W // _CHUNK
_N_GROUPS = _N_CHUNKS // _NB


def _emb_body(ids_hbm, table_hbm, out_hbm, idx_v, rows_v,
              g0, g1, g2, g3, w0, w1, w2, w3):
    gsems = (g0, g1, g2, g3)
    wsems = (w0, w1, w2, w3)
    wid = lax.axis_index("s") * _NC + lax.axis_index("c")
    base = wid * _ROWS_PER_W
    pltpu.sync_copy(ids_hbm.at[pl.ds(base, _ROWS_PER_W)], idx_v)

    def start_gather(j, b):
        pltpu.async_copy(
            table_hbm.at[idx_v.at[pl.ds(j * _CHUNK, _CHUNK)]],
            rows_v.at[b], gsems[b])

    def wait_gather(b):
        pltpu.make_async_copy(
            table_hbm.at[idx_v.at[pl.ds(0, _CHUNK)]],
            rows_v.at[b], gsems[b]).wait()

    def start_write(j, b):
        pltpu.async_copy(
            rows_v.at[b], out_hbm.at[pl.ds(base + j * _CHUNK, _CHUNK)],
            wsems[b])

    def wait_write(b):
        pltpu.make_async_copy(
            rows_v.at[b], out_hbm.at[pl.ds(base, _CHUNK)], wsems[b]).wait()

    for b in range(_NB):
        start_gather(b, b)

    def group(k, carry):
        for b in range(_NB):
            j = k * _NB + b
            wait_gather(b)
            start_write(j, b)
            wait_write(b)
            start_gather(j + _NB, b)
        return carry

    lax.fori_loop(0, _N_GROUPS - 1, group, 0)

    for b in range(_NB):
        j = (_N_GROUPS - 1) * _NB + b
        wait_gather(b)
        start_write(j, b)
        wait_write(b)


@jax.jit
def kernel(input_ids, table):
    ids = input_ids.reshape(-1).astype(jnp.int32)
    gather = pl.kernel(
        _emb_body,
        mesh=plsc.VectorSubcoreMesh(core_axis_name="c", subcore_axis_name="s"),
        out_type=jax.ShapeDtypeStruct((N_TOKENS, EMB_DIM), jnp.float32),
        scratch_types=[
            pltpu.VMEM((_ROWS_PER_W,), jnp.int32),
            pltpu.VMEM((_NB, _CHUNK, EMB_DIM), jnp.float32),
        ] + [pltpu.SemaphoreType.DMA] * (2 * _NB),
        compiler_params=pltpu.CompilerParams(use_tc_tiling_on_sc=False),
    )
    out = gather(ids, table)
    return out.reshape(input_ids.shape + (EMB_DIM,))


# speedup vs baseline: 1.1150x; 1.0010x over previous
import jax
import jax.numpy as jnp
from jax import lax
from jax.experimental import pallas as pl
from jax.experimental.pallas import tpu as pltpu
from jax.experimental.pallas import tpu_sc as plsc

VOCAB_ROWS = 1000000
EMB_DIM = 64
N_SEQ = 4096
SEQ_LEN = 200

_info = plsc.get_sparse_core_info()
_NC = _info.num_cores
_NS = _info.num_subcores
_NW = _NC * _NS
_SEQ_PER_W = N_SEQ // _NW
_NB = 4
_N_GROUPS = _SEQ_PER_W // _NB


def _emb_body(ids_hbm, table_hbm, out_hbm, idx_v, rows_v,
              g0, g1, g2, g3, w0, w1, w2, w3):
    gsems = (g0, g1, g2, g3)
    wsems = (w0, w1, w2, w3)
    wid = lax.axis_index("s") * _NC + lax.axis_index("c")
    base = wid * _SEQ_PER_W
    pltpu.sync_copy(ids_hbm.at[pl.ds(base, _SEQ_PER_W), :], idx_v)

    def start_gather(s, b):
        pltpu.async_copy(table_hbm.at[idx_v.at[s]], rows_v.at[b], gsems[b])

    def wait_gather(b):
        pltpu.make_async_copy(
            table_hbm.at[idx_v.at[0]], rows_v.at[b], gsems[b]).wait()

    def start_write(s, b):
        pltpu.async_copy(rows_v.at[b], out_hbm.at[base + s], wsems[b])

    def wait_write(b):
        pltpu.make_async_copy(
            rows_v.at[b], out_hbm.at[base], wsems[b]).wait()

    for b in range(_NB):
        start_gather(b, b)

    def group(k, carry):
        for b in range(_NB):
            s = k * _NB + b
            wait_gather(b)
            start_write(s, b)
            wait_write(b)
            start_gather(s + _NB, b)
        return carry

    lax.fori_loop(0, _N_GROUPS - 1, group, 0)

    for b in range(_NB):
        s = (_N_GROUPS - 1) * _NB + b
        wait_gather(b)
        start_write(s, b)
        wait_write(b)


@jax.jit
def kernel(input_ids, table):
    gather = pl.kernel(
        _emb_body,
        mesh=plsc.VectorSubcoreMesh(core_axis_name="c", subcore_axis_name="s"),
        out_type=jax.ShapeDtypeStruct((N_SEQ, SEQ_LEN, EMB_DIM), jnp.float32),
        scratch_types=[
            pltpu.VMEM((_SEQ_PER_W, SEQ_LEN), jnp.int32),
            pltpu.VMEM((_NB, SEQ_LEN, EMB_DIM), jnp.float32),
        ] + [pltpu.SemaphoreType.DMA] * (2 * _NB),
        compiler_params=pltpu.CompilerParams(use_tc_tiling_on_sc=False),
    )
    return gather(input_ids.astype(jnp.int32), table)
